# baseline (speedup 1.0000x reference)
import functools

import jax
import jax.numpy as jnp
from jax import lax
from jax.experimental import pallas as pl
from jax.experimental.pallas import tpu as pltpu
from jax.experimental.pallas import tpu_sc as plsc

_B = 16384
_D = 64
_V = 1000000


def _sc_gather_pairs(table2, idx2):
    info = plsc.get_sparse_core_info()
    NW = info.num_cores * info.num_subcores
    b_per_w = _B // NW
    mesh = plsc.VectorSubcoreMesh(core_axis_name="c", subcore_axis_name="s")

    @functools.partial(
        pl.kernel,
        mesh=mesh,
        out_type=jax.ShapeDtypeStruct((_B, 2 * _D), jnp.float32),
        scratch_types=[
            pltpu.VMEM((b_per_w,), jnp.int32),
            pltpu.VMEM((b_per_w, 2 * _D), jnp.float32),
            pltpu.SemaphoreType.DMA,
        ],
    )
    def k(table_hbm, idx_hbm, out_hbm, idx_v, rows_v, sem):
        wid = lax.axis_index("s") * info.num_cores + lax.axis_index("c")
        base = wid * b_per_w
        pltpu.sync_copy(idx_hbm.at[pl.ds(base, b_per_w)], idx_v)
        pltpu.async_copy(table_hbm.at[idx_v], rows_v, sem).wait()
        pltpu.sync_copy(rows_v, out_hbm.at[pl.ds(base, b_per_w)])

    return k(table2, idx2)


def _tc_mlp_sel(emb2, par, Wt, bias):
    BLK = 2048

    def body(emb_ref, par_ref, wt_ref, b_ref, out_ref):
        left = emb_ref[:, : _D]
        right = emb_ref[:, _D :]
        e = jnp.where(par_ref[...] > 0.5, right, left)
        h = jnp.maximum(e, 0.0)
        y = jnp.dot(h, wt_ref[...], preferred_element_type=jnp.float32)
        out_ref[...] = jnp.maximum(y + b_ref[...], 0.0)

    return pl.pallas_call(
        body,
        grid=(_B // BLK,),
        in_specs=[
            pl.BlockSpec((BLK, 2 * _D), lambda i: (i, 0)),
            pl.BlockSpec((BLK, 1), lambda i: (i, 0)),
            pl.BlockSpec((_D, _D), lambda i: (0, 0)),
            pl.BlockSpec((1, _D), lambda i: (0, 0)),
        ],
        out_specs=pl.BlockSpec((BLK, _D), lambda i: (i, 0)),
        out_shape=jax.ShapeDtypeStruct((_B, _D), jnp.float32),
    )(emb2, par, Wt, bias)


def kernel(task_indices, table, W, b):
    idx = task_indices.astype(jnp.int32)
    table2 = table.reshape(_V // 2, 2 * _D)
    emb2 = _sc_gather_pairs(table2, idx >> 1)
    par = (idx & 1).astype(jnp.float32).reshape(_B, 1)
    return _tc_mlp_sel(emb2, par, W.T, b.reshape(1, _D))


# speedup vs baseline: 1.0035x; 1.0035x over previous
import functools

import jax
import jax.numpy as jnp
from jax import lax
from jax.experimental import pallas as pl
from jax.experimental.pallas import tpu as pltpu
from jax.experimental.pallas import tpu_sc as plsc


def _sc_gather(table, idx, B, D):
    info = plsc.get_sparse_core_info()
    NW = info.num_cores * info.num_subcores
    b_per_w = B // NW
    mesh = plsc.VectorSubcoreMesh(core_axis_name="c", subcore_axis_name="s")

    @functools.partial(
        pl.kernel,
        mesh=mesh,
        compiler_params=pltpu.CompilerParams(use_tc_tiling_on_sc=False),
        out_type=jax.ShapeDtypeStruct((B, D), jnp.float32),
        scratch_types=[
            pltpu.VMEM((b_per_w,), jnp.int32),
            pltpu.VMEM((b_per_w, D), jnp.float32),
            pltpu.SemaphoreType.DMA,
        ],
    )
    def k(table_hbm, idx_hbm, out_hbm, idx_v, rows_v, sem):
        wid = lax.axis_index("s") * info.num_cores + lax.axis_index("c")
        base = wid * b_per_w
        pltpu.sync_copy(idx_hbm.at[pl.ds(base, b_per_w)], idx_v)
        pltpu.async_copy(table_hbm.at[idx_v], rows_v, sem).wait()
        pltpu.sync_copy(rows_v, out_hbm.at[pl.ds(base, b_per_w)])

    return k(table, idx)


def _tc_mlp(emb, Wt, bias):
    B, D = emb.shape
    BLK = 2048

    def body(emb_ref, wt_ref, b_ref, out_ref):
        h = jnp.maximum(emb_ref[...], 0.0)
        y = jnp.dot(h, wt_ref[...], preferred_element_type=jnp.float32)
        out_ref[...] = jnp.maximum(y + b_ref[...], 0.0)

    return pl.pallas_call(
        body,
        grid=(B // BLK,),
        in_specs=[
            pl.BlockSpec((BLK, D), lambda i: (i, 0)),
            pl.BlockSpec((D, D), lambda i: (0, 0)),
            pl.BlockSpec((1, D), lambda i: (0, 0)),
        ],
        out_specs=pl.BlockSpec((BLK, D), lambda i: (i, 0)),
        out_shape=jax.ShapeDtypeStruct((B, D), jnp.float32),
    )(emb, Wt, bias)


def kernel(task_indices, table, W, b):
    B = task_indices.shape[0]
    D = table.shape[1]
    idx = task_indices.astype(jnp.int32)
    emb = _sc_gather(table, idx, B, D)
    return _tc_mlp(emb, W.T, b.reshape(1, D))


# speedup vs baseline: 1.7158x; 1.7098x over previous
import functools

import jax
import jax.numpy as jnp
from jax import lax
from jax.experimental import pallas as pl
from jax.experimental.pallas import tpu as pltpu
from jax.experimental.pallas import tpu_sc as plsc


def _sc_gather(table, idx, B, D):
    info = plsc.get_sparse_core_info()
    NW = info.num_cores * info.num_subcores
    b_per_w = B // NW
    mesh = plsc.VectorSubcoreMesh(core_axis_name="c", subcore_axis_name="s")

    @functools.partial(
        pl.kernel,
        mesh=mesh,
        out_type=jax.ShapeDtypeStruct((B, D), jnp.float32),
        scratch_types=[
            pltpu.VMEM((b_per_w,), jnp.int32),
            pltpu.VMEM((b_per_w, D), jnp.float32),
            pltpu.SemaphoreType.DMA,
        ],
    )
    def k(table_hbm, idx_hbm, out_hbm, idx_v, rows_v, sem):
        wid = lax.axis_index("s") * info.num_cores + lax.axis_index("c")
        base = wid * b_per_w
        pltpu.sync_copy(idx_hbm.at[pl.ds(base, b_per_w)], idx_v)

        def issue(j, _):
            v = idx_v[pl.ds(j * 16, 16)]
            for lane in range(16):
                r = v[lane]
                pltpu.async_copy(table_hbm.at[r], rows_v.at[j * 16 + lane], sem)
            return 0

        lax.fori_loop(0, b_per_w // 16, issue, 0)

        def drain(i, _):
            pltpu.make_async_copy(table_hbm.at[0], rows_v.at[0], sem).wait()
            return 0

        lax.fori_loop(0, b_per_w, drain, 0, unroll=8)
        pltpu.sync_copy(rows_v, out_hbm.at[pl.ds(base, b_per_w)])

    return k(table, idx)


def _tc_mlp(emb, Wt, bias):
    B, D = emb.shape
    BLK = 2048

    def body(emb_ref, wt_ref, b_ref, out_ref):
        h = jnp.maximum(emb_ref[...], 0.0)
        y = jnp.dot(h, wt_ref[...], preferred_element_type=jnp.float32)
        out_ref[...] = jnp.maximum(y + b_ref[...], 0.0)

    return pl.pallas_call(
        body,
        grid=(B // BLK,),
        in_specs=[
            pl.BlockSpec((BLK, D), lambda i: (i, 0)),
            pl.BlockSpec((D, D), lambda i: (0, 0)),
            pl.BlockSpec((1, D), lambda i: (0, 0)),
        ],
        out_specs=pl.BlockSpec((BLK, D), lambda i: (i, 0)),
        out_shape=jax.ShapeDtypeStruct((B, D), jnp.float32),
    )(emb, Wt, bias)


def kernel(task_indices, table, W, b):
    B = task_indices.shape[0]
    D = table.shape[1]
    idx = task_indices.astype(jnp.int32)
    table_r = table.reshape(table.shape[0] * D).reshape(table.shape)
    emb = _sc_gather(table_r, idx, B, D)
    return _tc_mlp(emb, W.T, b.reshape(1, D))


# speedup vs baseline: 2.5044x; 1.4596x over previous
import functools

import jax
import jax.numpy as jnp
from jax import lax
from jax.experimental import pallas as pl
from jax.experimental.pallas import tpu as pltpu
from jax.experimental.pallas import tpu_sc as plsc


def _sc_gather(table, idx, B, D):
    info = plsc.get_sparse_core_info()
    NW = info.num_cores * info.num_subcores
    b_per_w = B // NW
    mesh = plsc.VectorSubcoreMesh(core_axis_name="c", subcore_axis_name="s")

    @functools.partial(
        pl.kernel,
        mesh=mesh,
        out_type=jax.ShapeDtypeStruct((B, D), jnp.float32),
        scratch_types=[
            pltpu.VMEM((b_per_w,), jnp.int32),
            pltpu.VMEM((b_per_w, D), jnp.float32),
            pltpu.SemaphoreType.DMA,
        ],
    )
    def k(table_hbm, idx_hbm, out_hbm, idx_v, rows_v, sem):
        wid = lax.axis_index("s") * info.num_cores + lax.axis_index("c")
        base = wid * b_per_w
        pltpu.sync_copy(idx_hbm.at[pl.ds(base, b_per_w)], idx_v)

        def issue(j, _):
            v = idx_v[pl.ds(j * 16, 16)]
            for lane in range(16):
                r = v[lane]
                pltpu.async_copy(
                    table_hbm.at[r >> 3, r & 7], rows_v.at[j * 16 + lane], sem
                )
            return 0

        lax.fori_loop(0, b_per_w // 16, issue, 0)

        def drain(i, _):
            pltpu.make_async_copy(table_hbm.at[0, 0], rows_v.at[0], sem).wait()
            return 0

        lax.fori_loop(0, b_per_w, drain, 0, unroll=8)
        pltpu.sync_copy(rows_v, out_hbm.at[pl.ds(base, b_per_w)])

    return k(table, idx)


def _tc_mlp(emb, Wt, bias):
    B, D = emb.shape
    BLK = 2048

    def body(emb_ref, wt_ref, b_ref, out_ref):
        h = jnp.maximum(emb_ref[...], 0.0)
        y = jnp.dot(h, wt_ref[...], preferred_element_type=jnp.float32)
        out_ref[...] = jnp.maximum(y + b_ref[...], 0.0)

    return pl.pallas_call(
        body,
        grid=(B // BLK,),
        in_specs=[
            pl.BlockSpec((BLK, D), lambda i: (i, 0)),
            pl.BlockSpec((D, D), lambda i: (0, 0)),
            pl.BlockSpec((1, D), lambda i: (0, 0)),
        ],
        out_specs=pl.BlockSpec((BLK, D), lambda i: (i, 0)),
        out_shape=jax.ShapeDtypeStruct((B, D), jnp.float32),
    )(emb, Wt, bias)


def kernel(task_indices, table, W, b):
    B = task_indices.shape[0]
    D = table.shape[1]
    idx = task_indices.astype(jnp.int32)
    table3 = table.reshape(table.shape[0] // 8, 8, D)
    emb = _sc_gather(table3, idx, B, D)
    return _tc_mlp(emb, W.T, b.reshape(1, D))


# speedup vs baseline: 2.5805x; 1.0304x over previous
import functools

import jax
import jax.numpy as jnp
from jax import lax
from jax.experimental import pallas as pl
from jax.experimental.pallas import tpu as pltpu
from jax.experimental.pallas import tpu_sc as plsc


def _sc_gather(table, idx, B, D):
    info = plsc.get_sparse_core_info()
    NW = info.num_cores * info.num_subcores
    b_per_w = B // NW
    mesh = plsc.VectorSubcoreMesh(core_axis_name="c", subcore_axis_name="s")

    @functools.partial(
        pl.kernel,
        mesh=mesh,
        out_type=jax.ShapeDtypeStruct((B, D), jnp.float32),
        scratch_types=[
            pltpu.VMEM((b_per_w,), jnp.int32),
            pltpu.VMEM((b_per_w, D), jnp.float32),
            pltpu.SemaphoreType.DMA,
        ],
    )
    def k(table_hbm, idx_hbm, out_hbm, idx_v, rows_v, sem):
        wid = lax.axis_index("s") * info.num_cores + lax.axis_index("c")
        base = wid * b_per_w
        pltpu.sync_copy(idx_hbm.at[pl.ds(base, b_per_w)], idx_v)

        def issue(j, _):
            v = idx_v[pl.ds(j * 16, 16)]
            for lane in range(16):
                r = v[lane]
                pltpu.async_copy(
                    table_hbm.at[r >> 3, r & 7], rows_v.at[j * 16 + lane], sem
                )
            return 0

        lax.fori_loop(0, b_per_w // 16, issue, 0)

        def drain(i, _):
            pltpu.make_async_copy(table_hbm.at[0, 0], rows_v.at[0], sem).wait()
            return 0

        lax.fori_loop(0, b_per_w, drain, 0, unroll=8)
        pltpu.sync_copy(rows_v, out_hbm.at[pl.ds(base, b_per_w)])

    return k(table, idx)


def _tc_mlp_t(emb, W, bias_col):
    B, D = emb.shape
    BLK = 2048

    def body(emb_ref, w_ref, b_ref, out_ref):
        h = jnp.maximum(emb_ref[...], 0.0)
        y = lax.dot_general(
            w_ref[...], h, (((1,), (1,)), ((), ())),
            preferred_element_type=jnp.float32,
        )
        out_ref[...] = jnp.maximum(y + b_ref[...], 0.0)

    return pl.pallas_call(
        body,
        grid=(B // BLK,),
        in_specs=[
            pl.BlockSpec((BLK, D), lambda i: (i, 0)),
            pl.BlockSpec((D, D), lambda i: (0, 0)),
            pl.BlockSpec((D, 1), lambda i: (0, 0)),
        ],
        out_specs=pl.BlockSpec((D, BLK), lambda i: (0, i)),
        out_shape=jax.ShapeDtypeStruct((D, B), jnp.float32),
    )(emb, W, bias_col)


def kernel(task_indices, table, W, b):
    B = task_indices.shape[0]
    D = table.shape[1]
    idx = task_indices.astype(jnp.int32)
    table3 = table.reshape(table.shape[0] // 8, 8, D)
    emb = _sc_gather(table3, idx, B, D)
    out_t = _tc_mlp_t(emb, W, b.reshape(D, 1))
    return out_t.T
